# Initial kernel scaffold; baseline (speedup 1.0000x reference)
import functools

import jax
import jax.numpy as jnp
from jax import lax
from jax.experimental import pallas as pl
from jax.experimental.pallas import tpu as pltpu
from jax.experimental.pallas import tpu_sc as plsc

_N = 10000
_E = 320000
_IN = 128
_OUT = 128
_R = 16
_B = 4

_TN = 400
_NT = _N // _TN

_NC = 2
_NS = 16
_NW = _NC * _NS
_EPW = _E // _NW
_W = 80
_KW = _EPW // _W
_RPS = _N // _NS


def _h_body(x_ref, bases_ref, coeff_ref, wselft_ref, b_ref, h_ref, self_ref):
    r = pl.program_id(1)
    c0 = coeff_ref[r, 0]
    c1 = coeff_ref[r, 1]
    c2 = coeff_ref[r, 2]
    c3 = coeff_ref[r, 3]
    w = (c0 * bases_ref[0] + c1 * bases_ref[1]
         + c2 * bases_ref[2] + c3 * bases_ref[3])
    x = x_ref[...]
    h_ref[0] = lax.dot_general(x, w, (((1,), (0,)), ((), ())),
                               preferred_element_type=jnp.float32)

    @pl.when(r == 0)
    def _():
        self_ref[...] = lax.dot_general(
            x, wselft_ref[...], (((1,), (0,)), ((), ())),
            preferred_element_type=jnp.float32) + b_ref[...]


def _h_pallas(x, bases, coeff, wself_t, b2d):
    return pl.pallas_call(
        _h_body,
        grid=(_NT, _R),
        in_specs=[
            pl.BlockSpec((_TN, _IN), lambda n, r: (n, 0)),
            pl.BlockSpec((_B, _IN, _OUT), lambda n, r: (0, 0, 0)),
            pl.BlockSpec(memory_space=pltpu.SMEM),
            pl.BlockSpec((_IN, _OUT), lambda n, r: (0, 0)),
            pl.BlockSpec((1, _OUT), lambda n, r: (0, 0)),
        ],
        out_specs=[
            pl.BlockSpec((1, _TN, _OUT), lambda n, r: (r, n, 0)),
            pl.BlockSpec((_TN, _OUT), lambda n, r: (n, 0)),
        ],
        out_shape=[
            jax.ShapeDtypeStruct((_R, _N, _OUT), jnp.float32),
            jax.ShapeDtypeStruct((_N, _OUT), jnp.float32),
        ],
    )(x, bases, coeff, wself_t, b2d)


def _sc_body(h_hbm, src_hbm, et_hbm, tgt_hbm, out_hbm,
             idx_v, et_v, tgt_v, gbuf, zbuf, acc_sh, sem):
    c = lax.axis_index("c")
    s = lax.axis_index("s")
    wid = s * _NC + c

    @pl.loop(0, _RPS // 5)
    def _(rr):
        for cc in range(_OUT // 16):
            zbuf[rr, pl.ds(cc * 16, 16)] = jnp.zeros((16,), jnp.float32)

    @pl.loop(0, 5)
    def _(k):
        pltpu.sync_copy(zbuf, acc_sh.at[pl.ds(s * _RPS + k * (_RPS // 5),
                                              _RPS // 5)])
    plsc.subcore_barrier()

    pltpu.sync_copy(src_hbm.at[wid], idx_v)
    pltpu.sync_copy(et_hbm.at[wid], et_v)
    pltpu.sync_copy(tgt_hbm.at[wid], tgt_v)

    @pl.loop(0, _KW)
    def _(j):
        for cc in range(_W // 16):
            sl = pl.ds(cc * 16, 16)
            idx_v[j, sl] = idx_v[j, sl] + et_v[j, sl] * _N

    @pl.loop(0, _KW)
    def _(j):
        pltpu.async_copy(h_hbm.at[idx_v.at[j]], gbuf, sem).wait()
        pltpu.sync_copy(gbuf, acc_sh.at[tgt_v.at[j]], add=True)

    plsc.subcore_barrier()

    pltpu.sync_copy(acc_sh.at[pl.ds(s * _RPS, _RPS)],
                    out_hbm.at[c].at[pl.ds(s * _RPS, _RPS)])


_sc_scatter = functools.partial(
    pl.kernel,
    out_type=jax.ShapeDtypeStruct((_NC, _N, _OUT), jnp.float32),
    mesh=plsc.VectorSubcoreMesh(core_axis_name="c", subcore_axis_name="s"),
    scratch_types=[
        pltpu.VMEM((_KW, _W), jnp.int32),
        pltpu.VMEM((_KW, _W), jnp.int32),
        pltpu.VMEM((_KW, _W), jnp.int32),
        pltpu.VMEM((_W, _OUT), jnp.float32),
        pltpu.VMEM((_RPS // 5, _OUT), jnp.float32),
        pltpu.VMEM_SHARED((_N, _OUT), jnp.float32),
        pltpu.SemaphoreType.DMA,
    ],
)(_sc_body)


def _relu_body(self_ref, acc_ref, out_ref):
    out_ref[...] = jnp.maximum(self_ref[...] + acc_ref[0] + acc_ref[1], 0.0)


def _relu_pallas(self_out, acc):
    return pl.pallas_call(
        _relu_body,
        grid=(_NT,),
        in_specs=[
            pl.BlockSpec((_TN, _OUT), lambda n: (n, 0)),
            pl.BlockSpec((_NC, _TN, _OUT), lambda n: (0, n, 0)),
        ],
        out_specs=pl.BlockSpec((_TN, _OUT), lambda n: (n, 0)),
        out_shape=jax.ShapeDtypeStruct((_N, _OUT), jnp.float32),
    )(self_out, acc)


def kernel(node_features, edge_index, edge_type, W_self_w, W_self_b,
           bases, coefficients):
    h, self_out = _h_pallas(node_features, bases, coefficients,
                            W_self_w.T, W_self_b.reshape(1, _OUT))
    src = edge_index[0].reshape(_NW, _KW, _W)
    tgt = edge_index[1].reshape(_NW, _KW, _W)
    et = edge_type.reshape(_NW, _KW, _W)
    acc = _sc_scatter(h.reshape(_R * _N, _OUT), src, et, tgt)
    return _relu_pallas(self_out, acc)


# speedup vs baseline: 2.4536x; 2.4536x over previous
import functools

import jax
import jax.numpy as jnp
from jax import lax
from jax.experimental import pallas as pl
from jax.experimental.pallas import tpu as pltpu
from jax.experimental.pallas import tpu_sc as plsc

_N = 10000
_E = 320000
_IN = 128
_OUT = 128
_R = 16
_B = 4

_TN = 400
_NT = _N // _TN

_NC = 2
_NS = 16
_NW = _NC * _NS
_EPW = _E // _NW
_W = 80
_KW = _EPW // _W
_SW = 25
_NSC = _KW // _SW
_NPAD = 10240
_RPS = _NPAD // _NS


def _h_body(x_ref, bases_ref, coeff_ref, wselft_ref, b_ref, h_ref, self_ref):
    r = pl.program_id(1)
    c0 = coeff_ref[r, 0]
    c1 = coeff_ref[r, 1]
    c2 = coeff_ref[r, 2]
    c3 = coeff_ref[r, 3]
    w = (c0 * bases_ref[0] + c1 * bases_ref[1]
         + c2 * bases_ref[2] + c3 * bases_ref[3])
    x = x_ref[...]
    h_ref[0] = lax.dot_general(x, w, (((1,), (0,)), ((), ())),
                               preferred_element_type=jnp.float32)

    @pl.when(r == 0)
    def _():
        self_ref[...] = lax.dot_general(
            x, wselft_ref[...], (((1,), (0,)), ((), ())),
            preferred_element_type=jnp.float32) + b_ref[...]


def _h_pallas(x, bases, coeff, wself_t, b2d):
    return pl.pallas_call(
        _h_body,
        grid=(_NT, _R),
        in_specs=[
            pl.BlockSpec((_TN, _IN), lambda n, r: (n, 0)),
            pl.BlockSpec((_B, _IN, _OUT), lambda n, r: (0, 0, 0)),
            pl.BlockSpec(memory_space=pltpu.SMEM),
            pl.BlockSpec((_IN, _OUT), lambda n, r: (0, 0)),
            pl.BlockSpec((1, _OUT), lambda n, r: (0, 0)),
        ],
        out_specs=[
            pl.BlockSpec((1, _TN, _OUT), lambda n, r: (r, n, 0)),
            pl.BlockSpec((_TN, _OUT), lambda n, r: (n, 0)),
        ],
        out_shape=[
            jax.ShapeDtypeStruct((_R, _N, _OUT), jnp.float32),
            jax.ShapeDtypeStruct((_N, _OUT), jnp.float32),
        ],
    )(x, bases, coeff, wself_t, b2d)


def _sc_body(h_hbm, src_hbm, et_hbm, tgt_hbm, out_hbm,
             idx_v, et_v, tgt_v, gbuf, acc_sh, sem):
    c = lax.axis_index("c")
    s = lax.axis_index("s")
    wid = s * _NC + c

    @pl.loop(0, _W)
    def _(rr):
        for cc in range(_OUT // 16):
            gbuf[rr, pl.ds(cc * 16, 16)] = jnp.zeros((16,), jnp.float32)

    @pl.loop(0, _RPS // _W)
    def _(k):
        pltpu.sync_copy(gbuf, acc_sh.at[pl.ds(s * _RPS + k * _W, _W)])
    plsc.subcore_barrier()

    @pl.loop(0, _NSC)
    def _(q):
        pltpu.sync_copy(src_hbm.at[wid, q], idx_v)
        pltpu.sync_copy(et_hbm.at[wid, q], et_v)
        pltpu.sync_copy(tgt_hbm.at[wid, q], tgt_v)

        @pl.loop(0, _SW)
        def _(j):
            for cc in range(_W // 16):
                sl = pl.ds(cc * 16, 16)
                idx_v[j, sl] = idx_v[j, sl] + et_v[j, sl] * _N

        @pl.loop(0, _SW)
        def _(j):
            pltpu.async_copy(h_hbm.at[idx_v.at[j]], gbuf, sem).wait()
            pltpu.sync_copy(gbuf, acc_sh.at[tgt_v.at[j]], add=True)

    plsc.subcore_barrier()

    pltpu.sync_copy(acc_sh.at[pl.ds(s * _RPS, _RPS)],
                    out_hbm.at[c].at[pl.ds(s * _RPS, _RPS)])


_sc_scatter = functools.partial(
    pl.kernel,
    out_type=jax.ShapeDtypeStruct((_NC, _NPAD, _OUT), jnp.float32),
    mesh=plsc.VectorSubcoreMesh(core_axis_name="c", subcore_axis_name="s"),
    scratch_types=[
        pltpu.VMEM((_SW, _W), jnp.int32),
        pltpu.VMEM((_SW, _W), jnp.int32),
        pltpu.VMEM((_SW, _W), jnp.int32),
        pltpu.VMEM((_W, _OUT), jnp.float32),
        pltpu.VMEM_SHARED((_NPAD, _OUT), jnp.float32),
        pltpu.SemaphoreType.DMA,
    ],
)(_sc_body)


def _relu_body(self_ref, acc_ref, out_ref):
    out_ref[...] = jnp.maximum(self_ref[...] + acc_ref[0] + acc_ref[1], 0.0)


def _relu_pallas(self_out, acc):
    return pl.pallas_call(
        _relu_body,
        grid=(_NT,),
        in_specs=[
            pl.BlockSpec((_TN, _OUT), lambda n: (n, 0)),
            pl.BlockSpec((_NC, _TN, _OUT), lambda n: (0, n, 0)),
        ],
        out_specs=pl.BlockSpec((_TN, _OUT), lambda n: (n, 0)),
        out_shape=jax.ShapeDtypeStruct((_N, _OUT), jnp.float32),
    )(self_out, acc)


def kernel(node_features, edge_index, edge_type, W_self_w, W_self_b,
           bases, coefficients):
    h, self_out = _h_pallas(node_features, bases, coefficients,
                            W_self_w.T, W_self_b.reshape(1, _OUT))
    src = edge_index[0].reshape(_NW, _NSC, _SW, _W)
    tgt = edge_index[1].reshape(_NW, _NSC, _SW, _W)
    et = edge_type.reshape(_NW, _NSC, _SW, _W)
    acc = _sc_scatter(h.reshape(_R * _N, _OUT), src, et, tgt)
    return _relu_pallas(self_out, acc)


# speedup vs baseline: 3.6635x; 1.4931x over previous
import functools

import jax
import jax.numpy as jnp
from jax import lax
from jax.experimental import pallas as pl
from jax.experimental.pallas import tpu as pltpu
from jax.experimental.pallas import tpu_sc as plsc

_N = 10000
_E = 320000
_IN = 128
_OUT = 128
_R = 16
_B = 4

_TN = 400
_NT = _N // _TN

_NC = 2
_NS = 16
_NW = _NC * _NS
_EPW = _E // _NW
_W = 80
_KW = _EPW // _W
_SW = 25
_NSC = _KW // _SW
_NPAD = 10240
_RPS = _NPAD // _NS


def _h_body(x_ref, bases_ref, coeff_ref, wselft_ref, b_ref, h_ref, self_ref,
            wbig_ref):
    n = pl.program_id(0)

    @pl.when(n == 0)
    def _():
        for r in range(_R):
            w = (coeff_ref[r, 0] * bases_ref[0]
                 + coeff_ref[r, 1] * bases_ref[1]
                 + coeff_ref[r, 2] * bases_ref[2]
                 + coeff_ref[r, 3] * bases_ref[3])
            wbig_ref[:, r * _OUT:(r + 1) * _OUT] = w.astype(jnp.bfloat16)
        wbig_ref[:, _R * _OUT:] = wselft_ref[...].astype(jnp.bfloat16)

    x = x_ref[...].astype(jnp.bfloat16)
    h = lax.dot_general(x, wbig_ref[...], (((1,), (0,)), ((), ())),
                        preferred_element_type=jnp.float32)
    for r in range(_R):
        h_ref[r] = h[:, r * _OUT:(r + 1) * _OUT]
    self_ref[...] = h[:, _R * _OUT:] + b_ref[...]


def _h_pallas(x, bases, coeff, wself_t, b2d):
    return pl.pallas_call(
        _h_body,
        grid=(_NT,),
        in_specs=[
            pl.BlockSpec((_TN, _IN), lambda n: (n, 0)),
            pl.BlockSpec((_B, _IN, _OUT), lambda n: (0, 0, 0)),
            pl.BlockSpec(memory_space=pltpu.SMEM),
            pl.BlockSpec((_IN, _OUT), lambda n: (0, 0)),
            pl.BlockSpec((1, _OUT), lambda n: (0, 0)),
        ],
        out_specs=[
            pl.BlockSpec((_R, _TN, _OUT), lambda n: (0, n, 0)),
            pl.BlockSpec((_TN, _OUT), lambda n: (n, 0)),
        ],
        out_shape=[
            jax.ShapeDtypeStruct((_R, _N, _OUT), jnp.float32),
            jax.ShapeDtypeStruct((_N, _OUT), jnp.float32),
        ],
        scratch_shapes=[
            pltpu.VMEM((_IN, (_R + 1) * _OUT), jnp.bfloat16),
        ],
    )(x, bases, coeff, wself_t, b2d)


def _sc_body(h_hbm, src_hbm, et_hbm, tgt_hbm, out_hbm,
             idx_v, et_v, tgt_v, gbuf, acc_sh, sem):
    c = lax.axis_index("c")
    s = lax.axis_index("s")
    wid = s * _NC + c

    @pl.loop(0, _W)
    def _(rr):
        for cc in range(_OUT // 16):
            gbuf[rr, pl.ds(cc * 16, 16)] = jnp.zeros((16,), jnp.float32)

    @pl.loop(0, _RPS // _W)
    def _(k):
        pltpu.sync_copy(gbuf, acc_sh.at[pl.ds(s * _RPS + k * _W, _W)])
    plsc.subcore_barrier()

    @pl.loop(0, _NSC)
    def _(q):
        pltpu.sync_copy(src_hbm.at[wid, q], idx_v)
        pltpu.sync_copy(et_hbm.at[wid, q], et_v)
        pltpu.sync_copy(tgt_hbm.at[wid, q], tgt_v)

        @pl.loop(0, _SW)
        def _(j):
            for cc in range(_W // 16):
                sl = pl.ds(cc * 16, 16)
                idx_v[j, sl] = idx_v[j, sl] + et_v[j, sl] * _N

        @pl.loop(0, _SW)
        def _(j):
            pltpu.async_copy(h_hbm.at[idx_v.at[j]], gbuf, sem).wait()
            pltpu.sync_copy(gbuf, acc_sh.at[tgt_v.at[j]], add=True)

    plsc.subcore_barrier()

    pltpu.sync_copy(acc_sh.at[pl.ds(s * _RPS, _RPS)],
                    out_hbm.at[c].at[pl.ds(s * _RPS, _RPS)])


_sc_scatter = functools.partial(
    pl.kernel,
    out_type=jax.ShapeDtypeStruct((_NC, _NPAD, _OUT), jnp.float32),
    mesh=plsc.VectorSubcoreMesh(core_axis_name="c", subcore_axis_name="s"),
    scratch_types=[
        pltpu.VMEM((_SW, _W), jnp.int32),
        pltpu.VMEM((_SW, _W), jnp.int32),
        pltpu.VMEM((_SW, _W), jnp.int32),
        pltpu.VMEM((_W, _OUT), jnp.float32),
        pltpu.VMEM_SHARED((_NPAD, _OUT), jnp.float32),
        pltpu.SemaphoreType.DMA,
    ],
)(_sc_body)


def _relu_body(self_ref, acc_ref, out_ref):
    out_ref[...] = jnp.maximum(self_ref[...] + acc_ref[0] + acc_ref[1], 0.0)


def _relu_pallas(self_out, acc):
    return pl.pallas_call(
        _relu_body,
        grid=(_NT,),
        in_specs=[
            pl.BlockSpec((_TN, _OUT), lambda n: (n, 0)),
            pl.BlockSpec((_NC, _TN, _OUT), lambda n: (0, n, 0)),
        ],
        out_specs=pl.BlockSpec((_TN, _OUT), lambda n: (n, 0)),
        out_shape=jax.ShapeDtypeStruct((_N, _OUT), jnp.float32),
    )(self_out, acc)


def kernel(node_features, edge_index, edge_type, W_self_w, W_self_b,
           bases, coefficients):
    h, self_out = _h_pallas(node_features, bases, coefficients,
                            W_self_w.T, W_self_b.reshape(1, _OUT))
    src = edge_index[0].reshape(_NW, _NSC, _SW, _W)
    tgt = edge_index[1].reshape(_NW, _NSC, _SW, _W)
    et = edge_type.reshape(_NW, _NSC, _SW, _W)
    acc = _sc_scatter(h.reshape(_R * _N, _OUT), src, et, tgt)
    return _relu_pallas(self_out, acc)


# speedup vs baseline: 5.0464x; 1.3775x over previous
import functools

import jax
import jax.numpy as jnp
from jax import lax
from jax.experimental import pallas as pl
from jax.experimental.pallas import tpu as pltpu
from jax.experimental.pallas import tpu_sc as plsc

_N = 10000
_E = 320000
_IN = 128
_OUT = 128
_R = 16
_B = 4

_TN = 400
_NT = _N // _TN

_NC = 2
_NS = 16
_NW = _NC * _NS
_EPW = _E // _NW
_W = 80
_KW = _EPW // _W
_SW = 25
_NSC = _KW // _SW
_NPAD = 10240
_RPS = _NPAD // _NS


def _h_body(x_ref, bases_ref, coeff_ref, wselft_ref, b_ref, h_ref, self_ref,
            wbig_ref):
    n = pl.program_id(0)

    @pl.when(n == 0)
    def _():
        for r in range(_R):
            w = (coeff_ref[r, 0] * bases_ref[0]
                 + coeff_ref[r, 1] * bases_ref[1]
                 + coeff_ref[r, 2] * bases_ref[2]
                 + coeff_ref[r, 3] * bases_ref[3])
            wbig_ref[:, r * _OUT:(r + 1) * _OUT] = w.astype(jnp.bfloat16)
        wbig_ref[:, _R * _OUT:] = wselft_ref[...].astype(jnp.bfloat16)

    x = x_ref[...].astype(jnp.bfloat16)
    h = lax.dot_general(x, wbig_ref[...], (((1,), (0,)), ((), ())),
                        preferred_element_type=jnp.float32)
    for r in range(_R):
        h_ref[r] = h[:, r * _OUT:(r + 1) * _OUT]
    self_ref[...] = h[:, _R * _OUT:] + b_ref[...]


def _h_pallas(x, bases, coeff, wself_t, b2d):
    return pl.pallas_call(
        _h_body,
        grid=(_NT,),
        in_specs=[
            pl.BlockSpec((_TN, _IN), lambda n: (n, 0)),
            pl.BlockSpec((_B, _IN, _OUT), lambda n: (0, 0, 0)),
            pl.BlockSpec(memory_space=pltpu.SMEM),
            pl.BlockSpec((_IN, _OUT), lambda n: (0, 0)),
            pl.BlockSpec((1, _OUT), lambda n: (0, 0)),
        ],
        out_specs=[
            pl.BlockSpec((_R, _TN, _OUT), lambda n: (0, n, 0)),
            pl.BlockSpec((_TN, _OUT), lambda n: (n, 0)),
        ],
        out_shape=[
            jax.ShapeDtypeStruct((_R, _N, _OUT), jnp.float32),
            jax.ShapeDtypeStruct((_N, _OUT), jnp.float32),
        ],
        scratch_shapes=[
            pltpu.VMEM((_IN, (_R + 1) * _OUT), jnp.bfloat16),
        ],
    )(x, bases, coeff, wself_t, b2d)


def _sc_body(h_hbm, src_hbm, et_hbm, tgt_hbm, out_hbm,
             idx_v, et_v, tgt_v, gbuf0, gbuf1, acc_sh, sem0, sem1):
    c = lax.axis_index("c")
    s = lax.axis_index("s")
    wid = s * _NC + c

    @pl.loop(0, _W)
    def _(rr):
        for cc in range(_OUT // 16):
            gbuf0[rr, pl.ds(cc * 16, 16)] = jnp.zeros((16,), jnp.float32)

    @pl.loop(0, _RPS // _W)
    def _(k):
        pltpu.sync_copy(gbuf0, acc_sh.at[pl.ds(s * _RPS + k * _W, _W)])
    plsc.subcore_barrier()

    @pl.loop(0, _NSC)
    def _(q):
        pltpu.sync_copy(src_hbm.at[wid, q], idx_v)
        pltpu.sync_copy(et_hbm.at[wid, q], et_v)
        pltpu.sync_copy(tgt_hbm.at[wid, q], tgt_v)

        @pl.loop(0, _SW)
        def _(j):
            for cc in range(_W // 16):
                sl = pl.ds(cc * 16, 16)
                idx_v[j, sl] = idx_v[j, sl] + et_v[j, sl] * _N

        pltpu.async_copy(h_hbm.at[idx_v.at[0]], gbuf0, sem0)

        @pl.loop(0, (_SW - 1) // 2)
        def _(k):
            j = 2 * k
            pltpu.async_copy(h_hbm.at[idx_v.at[j + 1]], gbuf1, sem1)
            pltpu.make_async_copy(h_hbm.at[idx_v.at[j]], gbuf0, sem0).wait()
            pltpu.sync_copy(gbuf0, acc_sh.at[tgt_v.at[j]], add=True)
            pltpu.async_copy(h_hbm.at[idx_v.at[j + 2]], gbuf0, sem0)
            pltpu.make_async_copy(h_hbm.at[idx_v.at[j + 1]], gbuf1, sem1).wait()
            pltpu.sync_copy(gbuf1, acc_sh.at[tgt_v.at[j + 1]], add=True)

        pltpu.make_async_copy(h_hbm.at[idx_v.at[_SW - 1]], gbuf0, sem0).wait()
        pltpu.sync_copy(gbuf0, acc_sh.at[tgt_v.at[_SW - 1]], add=True)

    plsc.subcore_barrier()

    pltpu.sync_copy(acc_sh.at[pl.ds(s * _RPS, _RPS)],
                    out_hbm.at[c].at[pl.ds(s * _RPS, _RPS)])


_sc_scatter = functools.partial(
    pl.kernel,
    out_type=jax.ShapeDtypeStruct((_NC, _NPAD, _OUT), jnp.float32),
    mesh=plsc.VectorSubcoreMesh(core_axis_name="c", subcore_axis_name="s"),
    scratch_types=[
        pltpu.VMEM((_SW, _W), jnp.int32),
        pltpu.VMEM((_SW, _W), jnp.int32),
        pltpu.VMEM((_SW, _W), jnp.int32),
        pltpu.VMEM((_W, _OUT), jnp.float32),
        pltpu.VMEM((_W, _OUT), jnp.float32),
        pltpu.VMEM_SHARED((_NPAD, _OUT), jnp.float32),
        pltpu.SemaphoreType.DMA,
        pltpu.SemaphoreType.DMA,
    ],
)(_sc_body)


def _relu_body(self_ref, acc_ref, out_ref):
    out_ref[...] = jnp.maximum(self_ref[...] + acc_ref[0] + acc_ref[1], 0.0)


def _relu_pallas(self_out, acc):
    return pl.pallas_call(
        _relu_body,
        grid=(_NT,),
        in_specs=[
            pl.BlockSpec((_TN, _OUT), lambda n: (n, 0)),
            pl.BlockSpec((_NC, _TN, _OUT), lambda n: (0, n, 0)),
        ],
        out_specs=pl.BlockSpec((_TN, _OUT), lambda n: (n, 0)),
        out_shape=jax.ShapeDtypeStruct((_N, _OUT), jnp.float32),
    )(self_out, acc)


def kernel(node_features, edge_index, edge_type, W_self_w, W_self_b,
           bases, coefficients):
    h, self_out = _h_pallas(node_features, bases, coefficients,
                            W_self_w.T, W_self_b.reshape(1, _OUT))
    src = edge_index[0].reshape(_NW, _NSC, _SW, _W)
    tgt = edge_index[1].reshape(_NW, _NSC, _SW, _W)
    et = edge_type.reshape(_NW, _NSC, _SW, _W)
    acc = _sc_scatter(h.reshape(_R * _N, _OUT), src, et, tgt)
    return _relu_pallas(self_out, acc)


# speedup vs baseline: 5.5497x; 1.0997x over previous
import functools

import jax
import jax.numpy as jnp
from jax import lax
from jax.experimental import pallas as pl
from jax.experimental.pallas import tpu as pltpu
from jax.experimental.pallas import tpu_sc as plsc

_N = 10000
_E = 320000
_IN = 128
_OUT = 128
_R = 16
_B = 4

_TN = 400
_NT = _N // _TN

_NC = 2
_NS = 16
_NW = _NC * _NS
_EPW = _E // _NW
_W = 80
_KW = _EPW // _W
_SW = 25
_NSC = _KW // _SW
_NPAD = 10240
_RPS = _NPAD // _NS


def _h_body(x_ref, bases_ref, coeff_ref, wselft_ref, b_ref, h_ref, self_ref,
            wbig_ref):
    n = pl.program_id(0)

    @pl.when(n == 0)
    def _():
        for r in range(_R):
            w = (coeff_ref[r, 0] * bases_ref[0]
                 + coeff_ref[r, 1] * bases_ref[1]
                 + coeff_ref[r, 2] * bases_ref[2]
                 + coeff_ref[r, 3] * bases_ref[3])
            wbig_ref[:, r * _OUT:(r + 1) * _OUT] = w.astype(jnp.bfloat16)
        wbig_ref[:, _R * _OUT:] = wselft_ref[...].astype(jnp.bfloat16)

    x = x_ref[...].astype(jnp.bfloat16)
    h = lax.dot_general(x, wbig_ref[...], (((1,), (0,)), ((), ())),
                        preferred_element_type=jnp.float32)
    for r in range(_R):
        h_ref[r] = h[:, r * _OUT:(r + 1) * _OUT]
    self_ref[...] = h[:, _R * _OUT:] + b_ref[...]


def _h_pallas(x, bases, coeff, wself_t, b2d):
    return pl.pallas_call(
        _h_body,
        grid=(_NT,),
        in_specs=[
            pl.BlockSpec((_TN, _IN), lambda n: (n, 0)),
            pl.BlockSpec((_B, _IN, _OUT), lambda n: (0, 0, 0)),
            pl.BlockSpec(memory_space=pltpu.SMEM),
            pl.BlockSpec((_IN, _OUT), lambda n: (0, 0)),
            pl.BlockSpec((1, _OUT), lambda n: (0, 0)),
        ],
        out_specs=[
            pl.BlockSpec((_R, _TN, _OUT), lambda n: (0, n, 0)),
            pl.BlockSpec((_TN, _OUT), lambda n: (n, 0)),
        ],
        out_shape=[
            jax.ShapeDtypeStruct((_R, _N, _OUT), jnp.float32),
            jax.ShapeDtypeStruct((_N, _OUT), jnp.float32),
        ],
        scratch_shapes=[
            pltpu.VMEM((_IN, (_R + 1) * _OUT), jnp.bfloat16),
        ],
    )(x, bases, coeff, wself_t, b2d)


def _sc_body(h_hbm, ei_hbm, et_hbm, out_hbm,
             idx_v, et_v, tgt_v, gbuf0, gbuf1, acc_sh, sem0, sem1):
    c = lax.axis_index("c")
    s = lax.axis_index("s")
    wid = s * _NC + c

    @pl.loop(0, _W)
    def _(rr):
        for cc in range(_OUT // 16):
            gbuf0[rr, pl.ds(cc * 16, 16)] = jnp.zeros((16,), jnp.float32)

    @pl.loop(0, _RPS // _W)
    def _(k):
        pltpu.sync_copy(gbuf0, acc_sh.at[pl.ds(s * _RPS + k * _W, _W)])
    plsc.subcore_barrier()

    @pl.loop(0, _NSC)
    def _(q):
        pltpu.sync_copy(ei_hbm.at[0, wid, q], idx_v)
        pltpu.sync_copy(et_hbm.at[wid, q], et_v)
        pltpu.sync_copy(ei_hbm.at[1, wid, q], tgt_v)

        @pl.loop(0, _SW)
        def _(j):
            for cc in range(_W // 16):
                sl = pl.ds(cc * 16, 16)
                idx_v[j, sl] = idx_v[j, sl] + et_v[j, sl] * _N

        pltpu.async_copy(h_hbm.at[idx_v.at[0]], gbuf0, sem0)

        @pl.loop(0, (_SW - 1) // 2)
        def _(k):
            j = 2 * k
            pltpu.async_copy(h_hbm.at[idx_v.at[j + 1]], gbuf1, sem1)
            pltpu.make_async_copy(h_hbm.at[idx_v.at[j]], gbuf0, sem0).wait()
            pltpu.sync_copy(gbuf0, acc_sh.at[tgt_v.at[j]], add=True)
            pltpu.async_copy(h_hbm.at[idx_v.at[j + 2]], gbuf0, sem0)
            pltpu.make_async_copy(h_hbm.at[idx_v.at[j + 1]], gbuf1, sem1).wait()
            pltpu.sync_copy(gbuf1, acc_sh.at[tgt_v.at[j + 1]], add=True)

        pltpu.make_async_copy(h_hbm.at[idx_v.at[_SW - 1]], gbuf0, sem0).wait()
        pltpu.sync_copy(gbuf0, acc_sh.at[tgt_v.at[_SW - 1]], add=True)

    plsc.subcore_barrier()

    pltpu.sync_copy(acc_sh.at[pl.ds(s * _RPS, _RPS)],
                    out_hbm.at[c].at[pl.ds(s * _RPS, _RPS)])


_sc_scatter = functools.partial(
    pl.kernel,
    out_type=jax.ShapeDtypeStruct((_NC, _NPAD, _OUT), jnp.float32),
    mesh=plsc.VectorSubcoreMesh(core_axis_name="c", subcore_axis_name="s"),
    scratch_types=[
        pltpu.VMEM((_SW, _W), jnp.int32),
        pltpu.VMEM((_SW, _W), jnp.int32),
        pltpu.VMEM((_SW, _W), jnp.int32),
        pltpu.VMEM((_W, _OUT), jnp.float32),
        pltpu.VMEM((_W, _OUT), jnp.float32),
        pltpu.VMEM_SHARED((_NPAD, _OUT), jnp.float32),
        pltpu.SemaphoreType.DMA,
        pltpu.SemaphoreType.DMA,
    ],
)(_sc_body)


def _relu_body(self_ref, acc_ref, out_ref):
    out_ref[...] = jnp.maximum(self_ref[...] + acc_ref[0] + acc_ref[1], 0.0)


_TR = 2000


def _relu_pallas(self_out, acc):
    return pl.pallas_call(
        _relu_body,
        grid=(_N // _TR,),
        in_specs=[
            pl.BlockSpec((_TR, _OUT), lambda n: (n, 0)),
            pl.BlockSpec((_NC, _TR, _OUT), lambda n: (0, n, 0)),
        ],
        out_specs=pl.BlockSpec((_TR, _OUT), lambda n: (n, 0)),
        out_shape=jax.ShapeDtypeStruct((_N, _OUT), jnp.float32),
    )(self_out, acc)


def kernel(node_features, edge_index, edge_type, W_self_w, W_self_b,
           bases, coefficients):
    h, self_out = _h_pallas(node_features, bases, coefficients,
                            W_self_w.T, W_self_b.reshape(1, _OUT))
    ei = edge_index.reshape(2, _NW, _NSC, _SW, _W)
    et = edge_type.reshape(_NW, _NSC, _SW, _W)
    acc = _sc_scatter(h.reshape(_R * _N, _OUT), ei, et)
    return _relu_pallas(self_out, acc)
